# baseline (device time: 57206 ns/iter reference)
import jax
import jax.numpy as jnp
from jax import lax
from jax.experimental import pallas as pl
from jax.experimental.pallas import tpu as pltpu

UNROLL = 8
C = 8


def kernel(ids, E):
    V, D = E.shape
    T = ids.shape[0]
    H = T // 2
    S = H // C
    ids_col = ids.reshape(T, 1)

    def body(
        ids_smem,
        ids_col_ref,
        e_hbm,
        out_ref,
        gather,
        zsend,
        zrecv,
        xsend,
        xrecv,
        gsems,
        zsend_sems,
        zrecv_sems,
        xsend_sems,
        xrecv_sems,
    ):
        x = lax.axis_index("x")
        y = lax.axis_index("y")
        z = lax.axis_index("z")
        peer_z = (x, y, 1 - z)
        peer_x = (1 - x, y, z)

        base = z * V
        h0 = x * H
        g0 = (1 - x) * H

        def issue_gather(c):
            def step(i, cnt):
                for u in range(UNROLL):
                    r = c * S + i * UNROLL + u
                    loc = ids_smem[h0 + r] - base
                    ok = (loc >= 0) & (loc < V)
                    idx = jnp.clip(loc, 0, V - 1)

                    @pl.when(ok)
                    def _():
                        pltpu.make_async_copy(
                            e_hbm.at[pl.ds(idx, 1), :],
                            gather.at[pl.ds(r, 1), :],
                            gsems.at[c],
                        ).start()

                    cnt = cnt + ok.astype(jnp.int32)
                return cnt

            return lax.fori_loop(0, S // UNROLL, step, jnp.int32(0))

        def wait_gather(c, n):
            def step(i, _):
                pltpu.make_async_copy(
                    e_hbm.at[pl.ds(0, 1), :],
                    gather.at[pl.ds(0, 1), :],
                    gsems.at[c],
                ).wait()
                return 0

            lax.fori_loop(0, n, step, 0)

        def z_rdma(c):
            sl = pl.ds(c * S, S)
            return pltpu.make_async_remote_copy(
                src_ref=zsend.at[sl],
                dst_ref=zrecv.at[sl],
                send_sem=zsend_sems.at[c],
                recv_sem=zrecv_sems.at[c],
                device_id=peer_z,
                device_id_type=pl.DeviceIdType.MESH,
            )

        def x_rdma(c):
            sl = pl.ds(c * S, S)
            return pltpu.make_async_remote_copy(
                src_ref=xsend.at[sl],
                dst_ref=xrecv.at[sl],
                send_sem=xsend_sems.at[c],
                recv_sem=xrecv_sems.at[c],
                device_id=peer_x,
                device_id_type=pl.DeviceIdType.MESH,
            )

        cnts = [None] * C
        cnts[0] = issue_gather(0)
        barrier = pltpu.get_barrier_semaphore()
        for peer in (peer_z, peer_x):
            pl.semaphore_signal(
                barrier, inc=1, device_id=peer, device_id_type=pl.DeviceIdType.MESH
            )
        pl.semaphore_wait(barrier, 2)
        for c in range(C):
            if c + 1 < C:
                cnts[c + 1] = issue_gather(c + 1)
            wait_gather(c, cnts[c])
            sl = pl.ds(c * S, S)
            idc = ids_col_ref[pl.ds(h0 + c * S, S), :]
            valid = (idc >= base) & (idc < base + V)
            zsend[sl] = jnp.where(valid, gather[sl], 0.0).astype(jnp.bfloat16)
            z_rdma(c).start()

        for c in range(C):
            z_rdma(c).wait_recv()
            sl = pl.ds(c * S, S)
            red = zsend[sl].astype(jnp.float32) + zrecv[sl].astype(jnp.float32)
            out_ref[pl.ds(h0 + c * S, S), :] = red
            xsend[sl] = red.astype(jnp.bfloat16)
            x_rdma(c).start()

        for c in range(C):
            x_rdma(c).wait_recv()
            sl = pl.ds(c * S, S)
            out_ref[pl.ds(g0 + c * S, S), :] = xrecv[sl].astype(jnp.float32)

        for c in range(C):
            z_rdma(c).wait_send()
            x_rdma(c).wait_send()

    return pl.pallas_call(
        body,
        out_shape=jax.ShapeDtypeStruct((T, D), jnp.float32),
        in_specs=[
            pl.BlockSpec(memory_space=pltpu.SMEM),
            pl.BlockSpec(memory_space=pltpu.VMEM),
            pl.BlockSpec(memory_space=pl.ANY),
        ],
        out_specs=pl.BlockSpec(memory_space=pltpu.VMEM),
        scratch_shapes=[
            pltpu.VMEM((H, D), jnp.float32),
            pltpu.VMEM((H, D), jnp.bfloat16),
            pltpu.VMEM((H, D), jnp.bfloat16),
            pltpu.VMEM((H, D), jnp.bfloat16),
            pltpu.VMEM((H, D), jnp.bfloat16),
            pltpu.SemaphoreType.DMA((C,)),
            pltpu.SemaphoreType.DMA((C,)),
            pltpu.SemaphoreType.DMA((C,)),
            pltpu.SemaphoreType.DMA((C,)),
            pltpu.SemaphoreType.DMA((C,)),
        ],
        compiler_params=pltpu.CompilerParams(collective_id=0),
    )(ids, ids_col, E)


# device time: 36658 ns/iter; 1.5605x vs baseline; 1.5605x over previous
import jax
import jax.numpy as jnp
from jax import lax
from jax.experimental import pallas as pl
from jax.experimental.pallas import tpu as pltpu

UNROLL = 8
C = 8


def kernel(ids, E):
    V, D = E.shape
    T = ids.shape[0]
    Q = T // 4
    S = Q // C
    F = S // 2

    x_out = lax.axis_index("x")
    y_out = lax.axis_index("y")
    z_out = lax.axis_index("z")
    h0_out = (2 * x_out + y_out) * Q
    my = lax.dynamic_slice(ids, (h0_out,), (Q,))
    loc = my - z_out * V
    valid = (loc >= 0) & (loc < V)
    locc = jnp.clip(loc, 0, V - 1)

    validc = valid.reshape(C, S)
    row = jnp.arange(S, dtype=jnp.int32)[None, :]
    packed = (
        ((~validc).astype(jnp.int32) << 30) | (row << 14) | locc.reshape(C, S)
    )
    packed = jnp.sort(packed, axis=1)
    src_sorted = packed & (V - 1)
    dst_sorted = ((packed >> 14) & (S - 1)) + (
        jnp.arange(C, dtype=jnp.int32) * S
    )[:, None]
    counts = validc.sum(axis=1).astype(jnp.int32)
    n_iters = (counts + UNROLL - 1) // UNROLL

    src_flat = src_sorted.reshape(Q)
    dst_flat = dst_sorted.reshape(Q)
    mask = valid.astype(jnp.float32).reshape(Q, 1)

    def body(
        src_smem,
        dst_smem,
        nit_smem,
        mask_ref,
        e_hbm,
        out_ref,
        gather,
        zsend,
        zrecv,
        rq,
        xrecv,
        yrecv,
        dxrecv,
        dyrecv,
        gsems,
        zs_sems,
        zr_sems,
        xs_sems,
        xr_sems,
        ys_sems,
        yr_sems,
        fxs_sems,
        dxr_sems,
        fys_sems,
        dyr_sems,
    ):
        x = lax.axis_index("x")
        y = lax.axis_index("y")
        z = lax.axis_index("z")
        peer_z = (x, y, 1 - z)
        peer_x = (1 - x, y, z)
        peer_y = (x, 1 - y, z)

        h0 = (2 * x + y) * Q
        qx0 = (2 * (1 - x) + y) * Q
        qy0 = (2 * x + (1 - y)) * Q
        qd0 = (2 * (1 - x) + (1 - y)) * Q

        def issue_gather(c):
            def step(i, _):
                for u in range(UNROLL):
                    j = c * S + i * UNROLL + u
                    pltpu.make_async_copy(
                        e_hbm.at[pl.ds(src_smem[j], 1), :],
                        gather.at[pl.ds(dst_smem[j], 1), :],
                        gsems.at[c],
                    ).start()
                return 0

            lax.fori_loop(0, nit_smem[c], step, 0)

        def wait_gather(c):
            def step(i, _):
                for u in range(UNROLL):
                    pltpu.make_async_copy(
                        e_hbm.at[pl.ds(0, 1), :],
                        gather.at[pl.ds(0, 1), :],
                        gsems.at[c],
                    ).wait()
                return 0

            lax.fori_loop(0, nit_smem[c], step, 0)

        def rdma(src, dst, ssem, rsem, peer, lo, n):
            sl = pl.ds(lo, n)
            return pltpu.make_async_remote_copy(
                src_ref=src.at[sl],
                dst_ref=dst.at[sl],
                send_sem=ssem,
                recv_sem=rsem,
                device_id=peer,
                device_id_type=pl.DeviceIdType.MESH,
            )

        def z_rdma(c):
            return rdma(zsend, zrecv, zs_sems.at[c], zr_sems.at[c], peer_z, c * S, S)

        def x_rdma(c):
            return rdma(rq, xrecv, xs_sems.at[c], xr_sems.at[c], peer_x, c * S, S)

        def y_rdma(c):
            return rdma(rq, yrecv, ys_sems.at[c], yr_sems.at[c], peer_y, c * S, S)

        def fx_rdma(c):
            return rdma(
                yrecv, dxrecv, fxs_sems.at[c], dxr_sems.at[c], peer_x, c * S, F
            )

        def fy_rdma(c):
            return rdma(
                xrecv, dyrecv, fys_sems.at[c], dyr_sems.at[c], peer_y, c * S + F, F
            )

        issue_gather(0)
        barrier = pltpu.get_barrier_semaphore()
        for peer in (peer_z, peer_x, peer_y):
            pl.semaphore_signal(
                barrier, inc=1, device_id=peer, device_id_type=pl.DeviceIdType.MESH
            )
        pl.semaphore_wait(barrier, 3)
        for c in range(C):
            if c + 1 < C:
                issue_gather(c + 1)
            wait_gather(c)
            sl = pl.ds(c * S, S)
            zsend[sl] = (gather[sl] * mask_ref[sl, :]).astype(jnp.bfloat16)
            z_rdma(c).start()

        for c in range(C):
            z_rdma(c).wait_recv()
            sl = pl.ds(c * S, S)
            red = zsend[sl].astype(jnp.float32) + zrecv[sl].astype(jnp.float32)
            out_ref[pl.ds(h0 + c * S, S), :] = red
            rq[sl] = red.astype(jnp.bfloat16)
            x_rdma(c).start()
            y_rdma(c).start()

        for c in range(C):
            x_rdma(c).wait_recv()
            out_ref[pl.ds(qx0 + c * S, S), :] = xrecv[pl.ds(c * S, S)].astype(
                jnp.float32
            )
            fy_rdma(c).start()
            y_rdma(c).wait_recv()
            out_ref[pl.ds(qy0 + c * S, S), :] = yrecv[pl.ds(c * S, S)].astype(
                jnp.float32
            )
            fx_rdma(c).start()

        for c in range(C):
            fx_rdma(c).wait_recv()
            out_ref[pl.ds(qd0 + c * S, F), :] = dxrecv[pl.ds(c * S, F)].astype(
                jnp.float32
            )
            fy_rdma(c).wait_recv()
            out_ref[pl.ds(qd0 + c * S + F, F), :] = dyrecv[
                pl.ds(c * S + F, F)
            ].astype(jnp.float32)

        for c in range(C):
            z_rdma(c).wait_send()
            x_rdma(c).wait_send()
            y_rdma(c).wait_send()
            fx_rdma(c).wait_send()
            fy_rdma(c).wait_send()

    return pl.pallas_call(
        body,
        out_shape=jax.ShapeDtypeStruct((T, D), jnp.float32),
        in_specs=[
            pl.BlockSpec(memory_space=pltpu.SMEM),
            pl.BlockSpec(memory_space=pltpu.SMEM),
            pl.BlockSpec(memory_space=pltpu.SMEM),
            pl.BlockSpec(memory_space=pltpu.VMEM),
            pl.BlockSpec(memory_space=pl.ANY),
        ],
        out_specs=pl.BlockSpec(memory_space=pltpu.VMEM),
        scratch_shapes=[
            pltpu.VMEM((Q, D), jnp.float32),
            pltpu.VMEM((Q, D), jnp.bfloat16),
            pltpu.VMEM((Q, D), jnp.bfloat16),
            pltpu.VMEM((Q, D), jnp.bfloat16),
            pltpu.VMEM((Q, D), jnp.bfloat16),
            pltpu.VMEM((Q, D), jnp.bfloat16),
            pltpu.VMEM((Q, D), jnp.bfloat16),
            pltpu.VMEM((Q, D), jnp.bfloat16),
            pltpu.SemaphoreType.DMA((C,)),
            pltpu.SemaphoreType.DMA((C,)),
            pltpu.SemaphoreType.DMA((C,)),
            pltpu.SemaphoreType.DMA((C,)),
            pltpu.SemaphoreType.DMA((C,)),
            pltpu.SemaphoreType.DMA((C,)),
            pltpu.SemaphoreType.DMA((C,)),
            pltpu.SemaphoreType.DMA((C,)),
            pltpu.SemaphoreType.DMA((C,)),
            pltpu.SemaphoreType.DMA((C,)),
            pltpu.SemaphoreType.DMA((C,)),
        ],
        compiler_params=pltpu.CompilerParams(collective_id=0),
    )(src_flat, dst_flat, n_iters, mask, E)
